# Initial kernel scaffold; baseline (speedup 1.0000x reference)
import functools

import jax
import jax.numpy as jnp
from jax import lax
from jax.experimental import pallas as pl
from jax.experimental.pallas import tpu as pltpu
from jax.experimental.pallas import tpu_sc as plsc

_B = 1024
_L = 200
_D = 64
_NC = 2
_NS = 16
_NW = _NC * _NS
_ROWS_PER_W = _B // _NW
_SPLIT = 104


def _make_embed():
    mesh = plsc.VectorSubcoreMesh(core_axis_name="c", subcore_axis_name="s")

    @functools.partial(
        pl.kernel,
        mesh=mesh,
        out_type=jax.ShapeDtypeStruct((_B, _L, _D), jnp.float32),
        scratch_types=[
            pltpu.VMEM((_L,), jnp.int32),
            pltpu.VMEM((_L, _D), jnp.float32),
            pltpu.VMEM((_L, _D), jnp.float32),
            pltpu.SemaphoreType.DMA,
        ],
    )
    def embed(x_hbm, tok_hbm, pos_hbm, out_hbm, idx_v, rows_v, pos_v, sem):
        wid = lax.axis_index("s") * _NC + lax.axis_index("c")
        base = wid * _ROWS_PER_W
        pltpu.sync_copy(pos_hbm.at[pl.ds(0, _L)], pos_v)

        def body(i, carry):
            row = base + i
            pltpu.sync_copy(x_hbm.at[row], idx_v)
            cp0 = pltpu.async_copy(
                tok_hbm.at[idx_v.at[pl.ds(0, _SPLIT)]],
                rows_v.at[pl.ds(0, _SPLIT)],
                sem,
            )
            cp1 = pltpu.async_copy(
                tok_hbm.at[idx_v.at[pl.ds(_SPLIT, _L - _SPLIT)]],
                rows_v.at[pl.ds(_SPLIT, _L - _SPLIT)],
                sem,
            )
            cp0.wait()
            cp1.wait()

            def add_body(r, c2):
                for c in range(_D // 16):
                    s = pl.ds(c * 16, 16)
                    rows_v[r, s] = rows_v[r, s] + pos_v[r, s]
                return c2

            lax.fori_loop(0, _L, add_body, 0)
            pltpu.sync_copy(rows_v, out_hbm.at[row])
            return carry

        lax.fori_loop(0, _ROWS_PER_W, body, 0)

    return embed


_embed = _make_embed()


def kernel(x, token_table, pos_table):
    return _embed(x.astype(jnp.int32), token_table, pos_table)


# speedup vs baseline: 2.5942x; 2.5942x over previous
import functools

import jax
import jax.numpy as jnp
from jax import lax
from jax.experimental import pallas as pl
from jax.experimental.pallas import tpu as pltpu
from jax.experimental.pallas import tpu_sc as plsc

_B = 1024
_L = 200
_D = 64
_NC = 2
_NS = 16
_NW = _NC * _NS
_ROWS_PER_W = _B // _NW
_SPLIT = 104


def _make_embed():
    mesh = plsc.VectorSubcoreMesh(core_axis_name="c", subcore_axis_name="s")

    @functools.partial(
        pl.kernel,
        mesh=mesh,
        out_type=jax.ShapeDtypeStruct((_B, _L, _D), jnp.float32),
        compiler_params=pltpu.CompilerParams(use_tc_tiling_on_sc=False),
        scratch_types=[
            pltpu.VMEM((_L,), jnp.int32),
            pltpu.VMEM((_L, _D), jnp.float32),
            pltpu.VMEM((_L, _D), jnp.float32),
            pltpu.SemaphoreType.DMA,
        ],
    )
    def embed(x_hbm, tok_hbm, pos_hbm, out_hbm, idx_v, rows_v, pos_v, sem):
        wid = lax.axis_index("s") * _NC + lax.axis_index("c")
        base = wid * _ROWS_PER_W
        pltpu.sync_copy(pos_hbm.at[pl.ds(0, _L)], pos_v)

        def body(i, carry):
            row = base + i
            pltpu.sync_copy(x_hbm.at[row], idx_v)
            cp0 = pltpu.async_copy(
                tok_hbm.at[idx_v.at[pl.ds(0, _SPLIT)]],
                rows_v.at[pl.ds(0, _SPLIT)],
                sem,
            )
            cp1 = pltpu.async_copy(
                tok_hbm.at[idx_v.at[pl.ds(_SPLIT, _L - _SPLIT)]],
                rows_v.at[pl.ds(_SPLIT, _L - _SPLIT)],
                sem,
            )
            cp0.wait()
            cp1.wait()

            def add_body(r, c2):
                for c in range(_D // 16):
                    s = pl.ds(c * 16, 16)
                    rows_v[r, s] = rows_v[r, s] + pos_v[r, s]
                return c2

            lax.fori_loop(0, _L, add_body, 0)
            pltpu.sync_copy(rows_v, out_hbm.at[row])
            return carry

        lax.fori_loop(0, _ROWS_PER_W, body, 0)

    return embed


_embed = _make_embed()


def kernel(x, token_table, pos_table):
    return _embed(x.astype(jnp.int32), token_table, pos_table)
